# baseline (speedup 1.0000x reference)
import jax
import jax.numpy as jnp
from jax import lax
from jax.experimental import pallas as pl
from jax.experimental.pallas import tpu as pltpu
from jax.experimental.pallas import tpu_sc as plsc

BATCH = 16384
EMB = 64
NROWS = 1000000

_Q = 16384
_BU = 4 * _Q
_GRID = (NROWS + _BU - 1) // _BU
_NP = _GRID * _Q

_NC = 2
_NS = 16
_NW = _NC * _NS
_BPW = BATCH // _NW
_CHUNK = 128
_NCHUNK = _BPW // _CHUNK


def _bf16_hi(v):
    return (v + 32768) & (-65536)


def _bf16_lo(v):
    return ((v + 32768) >> 16) & 65535


def _convert_body(x, out):
    xi = lax.bitcast_convert_type(x[...], jnp.int32)
    a = xi[:, :_Q]
    b = xi[:, _Q:2 * _Q]
    c = xi[:, 2 * _Q:3 * _Q]
    d = xi[:, 3 * _Q:]
    wab = _bf16_hi(a) | _bf16_lo(b)
    wcd = _bf16_hi(c) | _bf16_lo(d)
    w = jnp.concatenate([wab, wcd], axis=0)
    out[...] = w.T


@jax.jit
def _convert(tT):
    return pl.pallas_call(
        _convert_body,
        grid=(_GRID,),
        in_specs=[pl.BlockSpec((EMB, _BU), lambda g: (0, g))],
        out_specs=pl.BlockSpec((_Q, 2 * EMB), lambda g: (g, 0)),
        out_shape=jax.ShapeDtypeStruct((_NP, 2 * EMB), jnp.int32),
    )(tT)


def _sc_gather_body(idx_hbm, c_hbm, o_hbm, idx_v, rows_v, sem):
    wid = lax.axis_index("s") * _NC + lax.axis_index("c")
    base = wid * _BPW
    crow = wid * _NCHUNK
    pltpu.sync_copy(idx_hbm.at[pl.ds(crow, _NCHUNK)], idx_v)
    copies = []
    for j in range(_NCHUNK):
        copies.append(pltpu.async_copy(
            c_hbm.at[idx_v.at[j]],
            rows_v.at[pl.ds(j * _CHUNK, _CHUNK)], sem))
    for c in copies:
        c.wait()
    pltpu.sync_copy(rows_v, o_hbm.at[pl.ds(base, _BPW)])


@jax.jit
def _sc_gather(r2d, c):
    mesh = plsc.VectorSubcoreMesh(core_axis_name="c", subcore_axis_name="s")
    f = pl.kernel(
        _sc_gather_body,
        out_type=jax.ShapeDtypeStruct((BATCH, 2 * EMB), jnp.int32),
        mesh=mesh,
        scratch_types=[
            pltpu.VMEM((_NCHUNK, _CHUNK), jnp.int32),
            pltpu.VMEM((_BPW, 2 * EMB), jnp.int32),
            pltpu.SemaphoreType.DMA,
        ],
    )
    return f(r2d, c)


def _unpack(x, a2, e2):
    half = jnp.where(a2 == 0, x[:, :EMB], x[:, EMB:])
    bits = jnp.where(e2 == 0, half & (-65536), half << 16)
    return lax.bitcast_convert_type(bits, jnp.float32)


def _mlp_body(uo, vo, au, eu, ai, ei,
              w0u, w0v, b0, w1, b1, w2, b2, w3, b3, out):
    uf = _unpack(uo[...], au[...], eu[...])
    vf = _unpack(vo[...], ai[...], ei[...])
    h = uf @ w0u[...] + vf @ w0v[...] + b0[...]
    h = jnp.maximum(h, 0.0)
    h = jnp.maximum(h @ w1[...] + b1[...], 0.0)
    h = jnp.maximum(h @ w2[...] + b2[...], 0.0)
    out[...] = jnp.sum(h * w3[...], axis=1, keepdims=True) + b3[...]


_BLK = 4096


@jax.jit
def _mlp(uo, vo, au, eu, ai, ei, w0u, w0v, b0, w1, b1, w2, b2, w3, b3):
    nblk = BATCH // _BLK
    bcast = lambda i: (0, 0)
    row = lambda i: (i, 0)
    return pl.pallas_call(
        _mlp_body,
        grid=(nblk,),
        in_specs=[
            pl.BlockSpec((_BLK, 2 * EMB), row),
            pl.BlockSpec((_BLK, 2 * EMB), row),
            pl.BlockSpec((_BLK, 1), row),
            pl.BlockSpec((_BLK, 1), row),
            pl.BlockSpec((_BLK, 1), row),
            pl.BlockSpec((_BLK, 1), row),
            pl.BlockSpec((EMB, 32), bcast),
            pl.BlockSpec((EMB, 32), bcast),
            pl.BlockSpec((1, 32), bcast),
            pl.BlockSpec((32, 16), bcast),
            pl.BlockSpec((1, 16), bcast),
            pl.BlockSpec((16, 8), bcast),
            pl.BlockSpec((1, 8), bcast),
            pl.BlockSpec((1, 8), bcast),
            pl.BlockSpec((1, 1), bcast),
        ],
        out_specs=pl.BlockSpec((_BLK, 1), row),
        out_shape=jax.ShapeDtypeStruct((BATCH, 1), jnp.float32),
    )(uo, vo, au, eu, ai, ei, w0u, w0v, b0, w1, b1, w2, b2, w3, b3)


def kernel(u_id, i_id, user_table, item_table, W0, b0, W1, b1, W2, b2, W3, b3):
    u_id = u_id.astype(jnp.int32)
    i_id = i_id.astype(jnp.int32)
    u_r = (((u_id >> 16) << 14) | (u_id & 16383)).reshape(
        BATCH // _CHUNK, _CHUNK)
    i_r = (((i_id >> 16) << 14) | (i_id & 16383)).reshape(
        BATCH // _CHUNK, _CHUNK)
    cu = _convert(user_table.T)
    uo = _sc_gather(u_r, cu)
    ci = _convert(item_table.T)
    vo = _sc_gather(i_r, ci)
    out = _mlp(
        uo, vo,
        ((u_id >> 15) & 1).reshape(BATCH, 1),
        ((u_id >> 14) & 1).reshape(BATCH, 1),
        ((i_id >> 15) & 1).reshape(BATCH, 1),
        ((i_id >> 14) & 1).reshape(BATCH, 1),
        W0[:EMB], W0[EMB:], b0.reshape(1, -1),
        W1, b1.reshape(1, -1),
        W2, b2.reshape(1, -1),
        W3.reshape(1, -1), b3.reshape(1, 1),
    )
    return out[:, 0]


# speedup vs baseline: 1.0431x; 1.0431x over previous
import jax
import jax.numpy as jnp
from jax import lax
from jax.experimental import pallas as pl
from jax.experimental.pallas import tpu as pltpu
from jax.experimental.pallas import tpu_sc as plsc

BATCH = 16384
EMB = 64
NROWS = 1000000

_Q = 8192
_BU = 4 * _Q
_GRID = (NROWS + _BU - 1) // _BU
_NP = _GRID * _Q

_NC = 2
_NS = 16
_NW = _NC * _NS
_BPW = BATCH // _NW
_CHUNK = 128
_NCHUNK = _BPW // _CHUNK


def _bf16_hi(v):
    return (v + 32768) & (-65536)


def _bf16_lo(v):
    return ((v + 32768) >> 16) & 65535


def _convert_body(x, out):
    xi = lax.bitcast_convert_type(x[...], jnp.int32)
    a = xi[:, :_Q]
    b = xi[:, _Q:2 * _Q]
    c = xi[:, 2 * _Q:3 * _Q]
    d = xi[:, 3 * _Q:]
    wab = _bf16_hi(a) | _bf16_lo(b)
    wcd = _bf16_hi(c) | _bf16_lo(d)
    w = jnp.concatenate([wab, wcd], axis=0)
    out[...] = w.T


@jax.jit
def _convert(tT):
    return pl.pallas_call(
        _convert_body,
        grid=(_GRID,),
        in_specs=[pl.BlockSpec((EMB, _BU), lambda g: (0, g))],
        out_specs=pl.BlockSpec((_Q, 2 * EMB), lambda g: (g, 0)),
        out_shape=jax.ShapeDtypeStruct((_NP, 2 * EMB), jnp.int32),
    )(tT)


def _sc_gather_body(idx_hbm, c_hbm, o_hbm, idx_v, rows_v, sem):
    wid = lax.axis_index("s") * _NC + lax.axis_index("c")
    base = wid * _BPW
    crow = wid * _NCHUNK
    pltpu.sync_copy(idx_hbm.at[pl.ds(crow, _NCHUNK)], idx_v)
    copies = []
    for j in range(_NCHUNK):
        copies.append(pltpu.async_copy(
            c_hbm.at[idx_v.at[j]],
            rows_v.at[pl.ds(j * _CHUNK, _CHUNK)], sem))
    for c in copies:
        c.wait()
    pltpu.sync_copy(rows_v, o_hbm.at[pl.ds(base, _BPW)])


@jax.jit
def _sc_gather(r2d, c):
    mesh = plsc.VectorSubcoreMesh(core_axis_name="c", subcore_axis_name="s")
    f = pl.kernel(
        _sc_gather_body,
        out_type=jax.ShapeDtypeStruct((BATCH, 2 * EMB), jnp.int32),
        mesh=mesh,
        scratch_types=[
            pltpu.VMEM((_NCHUNK, _CHUNK), jnp.int32),
            pltpu.VMEM((_BPW, 2 * EMB), jnp.int32),
            pltpu.SemaphoreType.DMA,
        ],
    )
    return f(r2d, c)


def _unpack(x, ids):
    a2 = (ids >> 14) & 1
    e2 = (ids >> 13) & 1
    half = jnp.where(a2 == 0, x[:, :EMB], x[:, EMB:])
    bits = jnp.where(e2 == 0, half & (-65536), half << 16)
    return lax.bitcast_convert_type(bits, jnp.float32)


def _mlp_body(uo, vo, uid, iid,
              w0u, w0v, b0, w1, b1, w2, b2, w3, b3, out):
    uf = _unpack(uo[...], uid[...])
    vf = _unpack(vo[...], iid[...])
    h = uf @ w0u[...] + vf @ w0v[...] + b0[...]
    h = jnp.maximum(h, 0.0)
    h = jnp.maximum(h @ w1[...] + b1[...], 0.0)
    h = jnp.maximum(h @ w2[...] + b2[...], 0.0)
    out[...] = jnp.sum(h * w3[...], axis=1, keepdims=True) + b3[...]


_BLK = 8192


@jax.jit
def _mlp(uo, vo, uid, iid, w0u, w0v, b0, w1, b1, w2, b2, w3, b3):
    nblk = BATCH // _BLK
    bcast = lambda i: (0, 0)
    row = lambda i: (i, 0)
    return pl.pallas_call(
        _mlp_body,
        grid=(nblk,),
        in_specs=[
            pl.BlockSpec((_BLK, 2 * EMB), row),
            pl.BlockSpec((_BLK, 2 * EMB), row),
            pl.BlockSpec((_BLK, 1), row),
            pl.BlockSpec((_BLK, 1), row),
            pl.BlockSpec((EMB, 32), bcast),
            pl.BlockSpec((EMB, 32), bcast),
            pl.BlockSpec((1, 32), bcast),
            pl.BlockSpec((32, 16), bcast),
            pl.BlockSpec((1, 16), bcast),
            pl.BlockSpec((16, 8), bcast),
            pl.BlockSpec((1, 8), bcast),
            pl.BlockSpec((1, 8), bcast),
            pl.BlockSpec((1, 1), bcast),
        ],
        out_specs=pl.BlockSpec((_BLK, 1), row),
        out_shape=jax.ShapeDtypeStruct((BATCH, 1), jnp.float32),
    )(uo, vo, uid, iid, w0u, w0v, b0, w1, b1, w2, b2, w3, b3)


def kernel(u_id, i_id, user_table, item_table, W0, b0, W1, b1, W2, b2, W3, b3):
    u_id = u_id.astype(jnp.int32)
    i_id = i_id.astype(jnp.int32)
    u_r = (((u_id >> 15) << 13) | (u_id & 8191)).reshape(
        BATCH // _CHUNK, _CHUNK)
    i_r = (((i_id >> 15) << 13) | (i_id & 8191)).reshape(
        BATCH // _CHUNK, _CHUNK)
    cu = _convert(user_table.T)
    uo = _sc_gather(u_r, cu)
    ci = _convert(item_table.T)
    vo = _sc_gather(i_r, ci)
    out = _mlp(
        uo, vo,
        u_id.reshape(BATCH, 1), i_id.reshape(BATCH, 1),
        W0[:EMB], W0[EMB:], b0.reshape(1, -1),
        W1, b1.reshape(1, -1),
        W2, b2.reshape(1, -1),
        W3.reshape(1, -1), b3.reshape(1, 1),
    )
    return out[:, 0]
